# baseline (speedup 1.0000x reference)
import functools

import jax
import jax.numpy as jnp
from jax import lax
from jax.experimental import pallas as pl
from jax.experimental.pallas import tpu as pltpu
from jax.experimental.pallas import tpu_sc as plsc

B = 4096
H = 50
EMB = 32
BLK = 128
NC = 2
NS = 16
NW = NC * NS
BPW = B // NW
LANES = 16
ROW_VREGS = EMB // LANES


ZROWS = (VID_VOCAB := 1000000) // 4
ZPW = 7808
ZC = 128
NCH = ZPW // ZC
ZTAIL = ZROWS - NW * ZPW


def _sc_pack_table(vembT):
  mesh = plsc.VectorSubcoreMesh(core_axis_name="c", subcore_axis_name="s")

  @functools.partial(
      pl.kernel,
      mesh=mesh,
      compiler_params=pltpu.CompilerParams(needs_layout_passes=False),
      out_type=jax.ShapeDtypeStruct((ZROWS, BLK), jnp.float32),
      scratch_types=[
          pltpu.VMEM((EMB, 4 * (ZC + 16)), jnp.float32),
          pltpu.VMEM((EMB, 4 * (ZC + 16)), jnp.float32),
          pltpu.VMEM((ZC + 16, BLK), jnp.float32),
          pltpu.VMEM((ZC + 16, BLK), jnp.float32),
          pltpu.SemaphoreType.DMA,
          pltpu.SemaphoreType.DMA,
          pltpu.SemaphoreType.DMA,
      ],
  )
  def pack_kernel(vt_hbm, z_hbm, col_a, col_b, out_a, out_b,
                  sem_a, sem_b, sem_w):
    wid = lax.axis_index("s") * NC + lax.axis_index("c")
    zb = wid * ZPW
    vb = zb * 4
    cols = (col_a, col_b)
    outs = (out_a, out_b)
    sems = (sem_a, sem_b)

    def stage(c, buf, sem, n_rows):
      for j in range(EMB):
        pltpu.async_copy(
            vt_hbm.at[j, pl.ds(vb + c * (4 * ZC), 4 * n_rows)],
            buf.at[j, pl.ds(0, 4 * n_rows)], sem)

    def drain(c, buf, sem, n_rows):
      for j in range(EMB):
        pltpu.make_async_copy(
            vt_hbm.at[j, pl.ds(vb + c * (4 * ZC), 4 * n_rows)],
            buf.at[j, pl.ds(0, 4 * n_rows)], sem).wait()

    def transpose_chunk(buf, out, n_rows):
      def zbody(z, carry):
        for k in range(BLK // LANES):
          j0 = (k % 2) * LANES
          s = k // 2
          col = jnp.zeros((LANES,), jnp.int32) + (z * 4 + s)
          row = jax.lax.broadcasted_iota(jnp.int32, (LANES,), 0) + j0
          vals = plsc.load_gather(buf, [row, col])
          out[z, pl.ds(k * LANES, LANES)] = vals
        return carry

      lax.fori_loop(0, n_rows, zbody, 0)

    def write_issue(c, out):
      pltpu.async_copy(out.at[pl.ds(0, ZC), :],
                       z_hbm.at[pl.ds(zb + c * ZC, ZC), :], sem_w)

    def write_drain(c, out):
      pltpu.make_async_copy(out.at[pl.ds(0, ZC), :],
                            z_hbm.at[pl.ds(zb + c * ZC, ZC), :], sem_w).wait()

    stage(0, col_a, sem_a, ZC)
    stage(1, col_b, sem_b, ZC)

    def chunk_body(t, carry):
      c0 = t * 2
      drain(c0, col_a, sem_a, ZC)
      transpose_chunk(col_a, out_a, ZC)
      write_issue(c0, out_a)
      stage(c0 + 2, col_a, sem_a, ZC)
      c1 = c0 + 1
      drain(c1, col_b, sem_b, ZC)
      transpose_chunk(col_b, out_b, ZC)
      write_issue(c1, out_b)

      @pl.when(c1 + 2 < NCH)
      def _():
        stage(c1 + 2, col_b, sem_b, ZC)

      write_drain(c0, out_a)
      write_drain(c1, out_b)
      return carry

    lax.fori_loop(0, (NCH - 1) // 2, chunk_body, 0)
    drain(NCH - 1, col_a, sem_a, ZC)
    transpose_chunk(col_a, out_a, ZC)
    write_issue(NCH - 1, out_a)
    write_drain(NCH - 1, out_a)

    @pl.when(wid == NW - 1)
    def _tail():
      tb = NW * ZPW
      for j in range(EMB):
        pltpu.sync_copy(vt_hbm.at[j, pl.ds(tb * 4, 4 * ZTAIL)],
                        col_a.at[j, pl.ds(0, 4 * ZTAIL)])
      transpose_chunk(col_a, out_a, ZTAIL)
      pltpu.sync_copy(out_a.at[pl.ds(0, ZTAIL), :],
                      z_hbm.at[pl.ds(tb, ZTAIL), :])

  return pack_kernel(vembT)


def _sc_gather_pool(vid_g, wvt, region_g, cid_g, vemb4, remb4, cemb4):
  mesh = plsc.VectorSubcoreMesh(core_axis_name="c", subcore_axis_name="s")

  @functools.partial(
      pl.kernel,
      mesh=mesh,
      out_type=(
          jax.ShapeDtypeStruct((B, BLK), jnp.float32),
          jax.ShapeDtypeStruct((B, EMB), jnp.float32),
          jax.ShapeDtypeStruct((B, BLK), jnp.float32),
          jax.ShapeDtypeStruct((B, BLK), jnp.float32),
      ),
      scratch_types=[
          pltpu.VMEM((BPW,), jnp.int32),
          pltpu.VMEM((BPW,), jnp.int32),
          pltpu.VMEM((BPW,), jnp.int32),
          pltpu.VMEM((H, BPW), jnp.int32),
          pltpu.VMEM((BPW,), jnp.int32),
          pltpu.VMEM((BPW,), jnp.int32),
          pltpu.VMEM((BPW, BLK), jnp.float32),
          pltpu.VMEM((BPW, BLK), jnp.float32),
          pltpu.VMEM((BPW, BLK), jnp.float32),
          pltpu.VMEM((BPW, BLK), jnp.float32),
          pltpu.VMEM((BPW, BLK), jnp.float32),
          pltpu.VMEM((BPW, EMB), jnp.float32),
          pltpu.SemaphoreType.DMA,
          pltpu.SemaphoreType.DMA,
          pltpu.SemaphoreType.DMA,
          pltpu.SemaphoreType.DMA,
          pltpu.SemaphoreType.DMA,
      ],
  )
  def sc_kernel(vidg_hbm, wvt_hbm, regg_hbm, cidg_hbm,
                vemb_hbm, remb_hbm, cemb_hbm,
                out_v, out_p, out_r, out_c,
                vid_v, reg_v, cid_v, wvt_v, hg_a, hg_b,
                v_rows, r_rows, c_rows, buf_a, buf_b, acc,
                sem_a, sem_b, sem_v, sem_r, sem_c):
    wid = lax.axis_index("s") * NC + lax.axis_index("c")
    base = wid * BPW
    pltpu.sync_copy(vidg_hbm.at[pl.ds(base, BPW)], vid_v)
    pltpu.sync_copy(regg_hbm.at[pl.ds(base, BPW)], reg_v)
    pltpu.sync_copy(cidg_hbm.at[pl.ds(base, BPW)], cid_v)
    pltpu.sync_copy(wvt_hbm.at[:, pl.ds(base, BPW)], wvt_v)
    cp_v = pltpu.async_copy(vemb_hbm.at[vid_v], v_rows, sem_v)
    cp_r = pltpu.async_copy(remb_hbm.at[reg_v], r_rows, sem_r)
    cp_c = pltpu.async_copy(cemb_hbm.at[cid_v], c_rows, sem_c)

    def stage_blocks(h, hg):
      for q in range(BPW // LANES):
        hg[pl.ds(q * LANES, LANES)] = jnp.right_shift(
            wvt_v[h, pl.ds(q * LANES, LANES)], 2)

    def accumulate(buf, h):
      def add_chunk(q, carry):
        qb = pl.multiple_of(q * LANES, LANES)
        off_vec = jnp.bitwise_and(wvt_v[h, pl.ds(qb, LANES)], 3) * EMB
        for r in range(LANES):
          off = pl.multiple_of(off_vec[r], EMB)
          for j in range(ROW_VREGS):
            plsc.addupdate(acc.at[qb + r, pl.ds(j * LANES, LANES)],
                           buf[qb + r, pl.ds(off + j * LANES, LANES)])
        return carry

      lax.fori_loop(0, BPW // LANES, add_chunk, 0)

    def zero_body(b, carry):
      for j in range(ROW_VREGS):
        acc[b, pl.ds(j * LANES, LANES)] = jnp.zeros((LANES,), jnp.float32)
      return carry

    lax.fori_loop(0, BPW, zero_body, 0)
    stage_blocks(0, hg_a)
    pltpu.async_copy(vemb_hbm.at[hg_a], buf_a, sem_a)
    stage_blocks(1, hg_b)
    pltpu.async_copy(vemb_hbm.at[hg_b], buf_b, sem_b)

    def pair_body(t, carry):
      h = t * 2
      pltpu.make_async_copy(vemb_hbm.at[hg_a], buf_a, sem_a).wait()
      accumulate(buf_a, h)
      stage_blocks(h + 2, hg_a)
      pltpu.async_copy(vemb_hbm.at[hg_a], buf_a, sem_a)
      pltpu.make_async_copy(vemb_hbm.at[hg_b], buf_b, sem_b).wait()
      accumulate(buf_b, h + 1)
      stage_blocks(h + 3, hg_b)
      pltpu.async_copy(vemb_hbm.at[hg_b], buf_b, sem_b)
      return carry

    lax.fori_loop(0, H // 2 - 1, pair_body, 0)
    pltpu.make_async_copy(vemb_hbm.at[hg_a], buf_a, sem_a).wait()
    accumulate(buf_a, H - 2)
    pltpu.make_async_copy(vemb_hbm.at[hg_b], buf_b, sem_b).wait()
    accumulate(buf_b, H - 1)
    cp_v.wait()
    cp_r.wait()
    cp_c.wait()
    pltpu.sync_copy(v_rows, out_v.at[pl.ds(base, BPW)])
    pltpu.sync_copy(acc, out_p.at[pl.ds(base, BPW)])
    pltpu.sync_copy(r_rows, out_r.at[pl.ds(base, BPW)])
    pltpu.sync_copy(c_rows, out_c.at[pl.ds(base, BPW)])

  return sc_kernel(vid_g, wvt, region_g, cid_g, vemb4, remb4, cemb4)


def _select_sub(blocks_ref, sub_ref):
  blocks = blocks_ref[...]
  sub = sub_ref[...]
  out = jnp.zeros((blocks.shape[0], EMB), jnp.float32)
  for s in range(4):
    piece = blocks[:, s * EMB:(s + 1) * EMB]
    out = jnp.where(sub == s, piece, out)
  return out


def _mlp_body(v_ref, vs_ref, p_ref, r_ref, rs_ref, c_ref, cs_ref,
              w0, b0, w1, b1, w2, b2, wo, bo, out_ref):
  v = _select_sub(v_ref, vs_ref)
  r = _select_sub(r_ref, rs_ref)
  c = _select_sub(c_ref, cs_ref)
  feat = jnp.concatenate([v, p_ref[...], r, c], axis=-1)
  h = jnp.maximum(
      jnp.dot(feat, w0[...], preferred_element_type=jnp.float32) + b0[...], 0.0)
  h = jnp.maximum(
      jnp.dot(h, w1[...], preferred_element_type=jnp.float32) + b1[...], 0.0)
  h = jnp.maximum(
      jnp.dot(h, w2[...], preferred_element_type=jnp.float32) + b2[...], 0.0)
  out_ref[...] = jnp.dot(h, wo[...], preferred_element_type=jnp.float32) + bo[...]


def kernel(vid, watch_vids, region, cid, vemb, remb, cemb,
           W0, b0, W1, b1, W2, b2, Wo, bo):
  vid = vid.astype(jnp.int32)
  region = region.astype(jnp.int32)
  cid = cid.astype(jnp.int32)
  wvt = watch_vids.astype(jnp.int32).T
  vemb4 = _sc_pack_table(vemb.T)
  remb4 = remb.reshape(-1, BLK)
  cemb4 = cemb.reshape(-1, BLK)
  v4, pooled, r4, c4 = _sc_gather_pool(
      vid >> 2, wvt, region >> 2, cid >> 2, vemb4, remb4, cemb4)
  logit = pl.pallas_call(
      _mlp_body,
      out_shape=jax.ShapeDtypeStruct((B, 1), jnp.float32),
  )(v4, (vid & 3).reshape(B, 1), pooled,
    r4, (region & 3).reshape(B, 1), c4, (cid & 3).reshape(B, 1),
    W0, b0.reshape(1, -1), W1, b1.reshape(1, -1),
    W2, b2.reshape(1, -1), Wo, bo.reshape(1, -1))
  return logit


# speedup vs baseline: 1.1835x; 1.1835x over previous
import functools

import jax
import jax.numpy as jnp
from jax import lax
from jax.experimental import pallas as pl
from jax.experimental.pallas import tpu as pltpu
from jax.experimental.pallas import tpu_sc as plsc

B = 4096
H = 50
EMB = 32
BLK = 128
NC = 2
NS = 16
NW = NC * NS
BPW = B // NW
LANES = 16
ROW_VREGS = EMB // LANES


ZROWS = (VID_VOCAB := 1000000) // 4
ZPW = 7808
ZC = 128
NCH = ZPW // ZC
ZTAIL = ZROWS - NW * ZPW


def _sc_pack_table(vembT):
  mesh = plsc.VectorSubcoreMesh(core_axis_name="c", subcore_axis_name="s")

  @functools.partial(
      pl.kernel,
      mesh=mesh,
      compiler_params=pltpu.CompilerParams(needs_layout_passes=False),
      out_type=jax.ShapeDtypeStruct((ZROWS, BLK), jnp.float32),
      scratch_types=[
          pltpu.VMEM((EMB, 4 * (ZC + 16)), jnp.float32),
          pltpu.VMEM((EMB, 4 * (ZC + 16)), jnp.float32),
          pltpu.VMEM((ZC + 16, BLK), jnp.float32),
          pltpu.VMEM((ZC + 16, BLK), jnp.float32),
          pltpu.SemaphoreType.DMA,
          pltpu.SemaphoreType.DMA,
          pltpu.SemaphoreType.DMA,
      ],
  )
  def pack_kernel(vt_hbm, z_hbm, col_a, col_b, out_a, out_b,
                  sem_a, sem_b, sem_w):
    wid = lax.axis_index("s") * NC + lax.axis_index("c")
    zb = wid * ZPW
    vb = zb * 4
    cols = (col_a, col_b)
    outs = (out_a, out_b)
    sems = (sem_a, sem_b)

    def stage(c, buf, sem, n_rows):
      for j in range(EMB):
        pltpu.async_copy(
            vt_hbm.at[j, pl.ds(vb + c * (4 * ZC), 4 * n_rows)],
            buf.at[j, pl.ds(0, 4 * n_rows)], sem)

    def drain(c, buf, sem, n_rows):
      for j in range(EMB):
        pltpu.make_async_copy(
            vt_hbm.at[j, pl.ds(vb + c * (4 * ZC), 4 * n_rows)],
            buf.at[j, pl.ds(0, 4 * n_rows)], sem).wait()

    lane = jax.lax.broadcasted_iota(jnp.int32, (LANES,), 0)
    row_pat = jnp.right_shift(lane, 2)
    col_pat = jnp.bitwise_and(lane, 3) * EMB

    def transpose_chunk(buf, out, n_rows):
      def qbody(q, carry):
        q16 = pl.multiple_of(q * LANES, LANES)
        rows = row_pat + q * 4
        for j in range(EMB):
          vals = buf[j, pl.ds(q16, LANES)]
          plsc.store_scatter(out, [rows, col_pat + j], vals)
        return carry

      lax.fori_loop(0, n_rows * 4 // LANES, qbody, 0)

    def write_issue(c, out):
      pltpu.async_copy(out.at[pl.ds(0, ZC), :],
                       z_hbm.at[pl.ds(zb + c * ZC, ZC), :], sem_w)

    def write_drain(c, out):
      pltpu.make_async_copy(out.at[pl.ds(0, ZC), :],
                            z_hbm.at[pl.ds(zb + c * ZC, ZC), :], sem_w).wait()

    stage(0, col_a, sem_a, ZC)
    stage(1, col_b, sem_b, ZC)

    def chunk_body(t, carry):
      c0 = t * 2
      drain(c0, col_a, sem_a, ZC)
      transpose_chunk(col_a, out_a, ZC)
      write_issue(c0, out_a)
      stage(c0 + 2, col_a, sem_a, ZC)
      c1 = c0 + 1
      drain(c1, col_b, sem_b, ZC)
      transpose_chunk(col_b, out_b, ZC)
      write_issue(c1, out_b)

      @pl.when(c1 + 2 < NCH)
      def _():
        stage(c1 + 2, col_b, sem_b, ZC)

      write_drain(c0, out_a)
      write_drain(c1, out_b)
      return carry

    lax.fori_loop(0, (NCH - 1) // 2, chunk_body, 0)
    drain(NCH - 1, col_a, sem_a, ZC)
    transpose_chunk(col_a, out_a, ZC)
    write_issue(NCH - 1, out_a)
    write_drain(NCH - 1, out_a)

    @pl.when(wid == NW - 1)
    def _tail():
      tb = NW * ZPW
      for j in range(EMB):
        pltpu.sync_copy(vt_hbm.at[j, pl.ds(tb * 4, 4 * ZTAIL)],
                        col_a.at[j, pl.ds(0, 4 * ZTAIL)])
      transpose_chunk(col_a, out_a, ZTAIL)
      pltpu.sync_copy(out_a.at[pl.ds(0, ZTAIL), :],
                      z_hbm.at[pl.ds(tb, ZTAIL), :])

  return pack_kernel(vembT)


def _sc_gather_pool(vid_g, wvt, region_g, cid_g, vemb4, remb4, cemb4):
  mesh = plsc.VectorSubcoreMesh(core_axis_name="c", subcore_axis_name="s")

  @functools.partial(
      pl.kernel,
      mesh=mesh,
      out_type=(
          jax.ShapeDtypeStruct((B, BLK), jnp.float32),
          jax.ShapeDtypeStruct((B, EMB), jnp.float32),
          jax.ShapeDtypeStruct((B, BLK), jnp.float32),
          jax.ShapeDtypeStruct((B, BLK), jnp.float32),
      ),
      scratch_types=[
          pltpu.VMEM((BPW,), jnp.int32),
          pltpu.VMEM((BPW,), jnp.int32),
          pltpu.VMEM((BPW,), jnp.int32),
          pltpu.VMEM((H, BPW), jnp.int32),
          pltpu.VMEM((BPW,), jnp.int32),
          pltpu.VMEM((BPW,), jnp.int32),
          pltpu.VMEM((BPW, BLK), jnp.float32),
          pltpu.VMEM((BPW, BLK), jnp.float32),
          pltpu.VMEM((BPW, BLK), jnp.float32),
          pltpu.VMEM((BPW, BLK), jnp.float32),
          pltpu.VMEM((BPW, BLK), jnp.float32),
          pltpu.VMEM((BPW, EMB), jnp.float32),
          pltpu.SemaphoreType.DMA,
          pltpu.SemaphoreType.DMA,
          pltpu.SemaphoreType.DMA,
          pltpu.SemaphoreType.DMA,
          pltpu.SemaphoreType.DMA,
      ],
  )
  def sc_kernel(vidg_hbm, wvt_hbm, regg_hbm, cidg_hbm,
                vemb_hbm, remb_hbm, cemb_hbm,
                out_v, out_p, out_r, out_c,
                vid_v, reg_v, cid_v, wvt_v, hg_a, hg_b,
                v_rows, r_rows, c_rows, buf_a, buf_b, acc,
                sem_a, sem_b, sem_v, sem_r, sem_c):
    wid = lax.axis_index("s") * NC + lax.axis_index("c")
    base = wid * BPW
    pltpu.sync_copy(vidg_hbm.at[pl.ds(base, BPW)], vid_v)
    pltpu.sync_copy(regg_hbm.at[pl.ds(base, BPW)], reg_v)
    pltpu.sync_copy(cidg_hbm.at[pl.ds(base, BPW)], cid_v)
    pltpu.sync_copy(wvt_hbm.at[:, pl.ds(base, BPW)], wvt_v)
    cp_v = pltpu.async_copy(vemb_hbm.at[vid_v], v_rows, sem_v)
    cp_r = pltpu.async_copy(remb_hbm.at[reg_v], r_rows, sem_r)
    cp_c = pltpu.async_copy(cemb_hbm.at[cid_v], c_rows, sem_c)

    def stage_blocks(h, hg):
      for q in range(BPW // LANES):
        hg[pl.ds(q * LANES, LANES)] = jnp.right_shift(
            wvt_v[h, pl.ds(q * LANES, LANES)], 2)

    def accumulate(buf, h):
      def add_chunk(q, carry):
        qb = pl.multiple_of(q * LANES, LANES)
        off_vec = jnp.bitwise_and(wvt_v[h, pl.ds(qb, LANES)], 3) * EMB
        for r in range(LANES):
          off = pl.multiple_of(off_vec[r], EMB)
          for j in range(ROW_VREGS):
            plsc.addupdate(acc.at[qb + r, pl.ds(j * LANES, LANES)],
                           buf[qb + r, pl.ds(off + j * LANES, LANES)])
        return carry

      lax.fori_loop(0, BPW // LANES, add_chunk, 0)

    def zero_body(b, carry):
      for j in range(ROW_VREGS):
        acc[b, pl.ds(j * LANES, LANES)] = jnp.zeros((LANES,), jnp.float32)
      return carry

    lax.fori_loop(0, BPW, zero_body, 0)
    stage_blocks(0, hg_a)
    pltpu.async_copy(vemb_hbm.at[hg_a], buf_a, sem_a)
    stage_blocks(1, hg_b)
    pltpu.async_copy(vemb_hbm.at[hg_b], buf_b, sem_b)

    def pair_body(t, carry):
      h = t * 2
      pltpu.make_async_copy(vemb_hbm.at[hg_a], buf_a, sem_a).wait()
      accumulate(buf_a, h)
      stage_blocks(h + 2, hg_a)
      pltpu.async_copy(vemb_hbm.at[hg_a], buf_a, sem_a)
      pltpu.make_async_copy(vemb_hbm.at[hg_b], buf_b, sem_b).wait()
      accumulate(buf_b, h + 1)
      stage_blocks(h + 3, hg_b)
      pltpu.async_copy(vemb_hbm.at[hg_b], buf_b, sem_b)
      return carry

    lax.fori_loop(0, H // 2 - 1, pair_body, 0)
    pltpu.make_async_copy(vemb_hbm.at[hg_a], buf_a, sem_a).wait()
    accumulate(buf_a, H - 2)
    pltpu.make_async_copy(vemb_hbm.at[hg_b], buf_b, sem_b).wait()
    accumulate(buf_b, H - 1)
    cp_v.wait()
    cp_r.wait()
    cp_c.wait()
    pltpu.sync_copy(v_rows, out_v.at[pl.ds(base, BPW)])
    pltpu.sync_copy(acc, out_p.at[pl.ds(base, BPW)])
    pltpu.sync_copy(r_rows, out_r.at[pl.ds(base, BPW)])
    pltpu.sync_copy(c_rows, out_c.at[pl.ds(base, BPW)])

  return sc_kernel(vid_g, wvt, region_g, cid_g, vemb4, remb4, cemb4)


def _select_sub(blocks_ref, sub_ref):
  blocks = blocks_ref[...]
  sub = sub_ref[...]
  out = jnp.zeros((blocks.shape[0], EMB), jnp.float32)
  for s in range(4):
    piece = blocks[:, s * EMB:(s + 1) * EMB]
    out = jnp.where(sub == s, piece, out)
  return out


def _mlp_body(v_ref, vs_ref, p_ref, r_ref, rs_ref, c_ref, cs_ref,
              w0, b0, w1, b1, w2, b2, wo, bo, out_ref):
  v = _select_sub(v_ref, vs_ref)
  r = _select_sub(r_ref, rs_ref)
  c = _select_sub(c_ref, cs_ref)
  feat = jnp.concatenate([v, p_ref[...], r, c], axis=-1)
  h = jnp.maximum(
      jnp.dot(feat, w0[...], preferred_element_type=jnp.float32) + b0[...], 0.0)
  h = jnp.maximum(
      jnp.dot(h, w1[...], preferred_element_type=jnp.float32) + b1[...], 0.0)
  h = jnp.maximum(
      jnp.dot(h, w2[...], preferred_element_type=jnp.float32) + b2[...], 0.0)
  out_ref[...] = jnp.dot(h, wo[...], preferred_element_type=jnp.float32) + bo[...]


def kernel(vid, watch_vids, region, cid, vemb, remb, cemb,
           W0, b0, W1, b1, W2, b2, Wo, bo):
  vid = vid.astype(jnp.int32)
  region = region.astype(jnp.int32)
  cid = cid.astype(jnp.int32)
  wvt = watch_vids.astype(jnp.int32).T
  vemb4 = _sc_pack_table(vemb.T)
  remb4 = remb.reshape(-1, BLK)
  cemb4 = cemb.reshape(-1, BLK)
  v4, pooled, r4, c4 = _sc_gather_pool(
      vid >> 2, wvt, region >> 2, cid >> 2, vemb4, remb4, cemb4)
  logit = pl.pallas_call(
      _mlp_body,
      out_shape=jax.ShapeDtypeStruct((B, 1), jnp.float32),
  )(v4, (vid & 3).reshape(B, 1), pooled,
    r4, (region & 3).reshape(B, 1), c4, (cid & 3).reshape(B, 1),
    W0, b0.reshape(1, -1), W1, b1.reshape(1, -1),
    W2, b2.reshape(1, -1), Wo, bo.reshape(1, -1))
  return logit


# speedup vs baseline: 1.5579x; 1.3163x over previous
import functools

import jax
import jax.numpy as jnp
from jax import lax
from jax.experimental import pallas as pl
from jax.experimental.pallas import tpu as pltpu
from jax.experimental.pallas import tpu_sc as plsc

B = 4096
H = 50
EMB = 32
BLK = 128
NC = 2
NS = 16
NW = NC * NS
BPW = B // NW
LANES = 16
ROW_VREGS = EMB // LANES


def _sc_gather_pool(vid_g, wvt, region_g, cid_g, vemb4, remb4, cemb4):
  mesh = plsc.VectorSubcoreMesh(core_axis_name="c", subcore_axis_name="s")

  @functools.partial(
      pl.kernel,
      mesh=mesh,
      compiler_params=pltpu.CompilerParams(use_tc_tiling_on_sc=True),
      out_type=(
          jax.ShapeDtypeStruct((B, BLK), jnp.float32),
          jax.ShapeDtypeStruct((B, EMB), jnp.float32),
          jax.ShapeDtypeStruct((B, BLK), jnp.float32),
          jax.ShapeDtypeStruct((B, BLK), jnp.float32),
      ),
      scratch_types=[
          pltpu.VMEM((BPW,), jnp.int32),
          pltpu.VMEM((BPW,), jnp.int32),
          pltpu.VMEM((BPW,), jnp.int32),
          pltpu.VMEM((H, BPW), jnp.int32),
          pltpu.VMEM((BPW,), jnp.int32),
          pltpu.VMEM((BPW,), jnp.int32),
          pltpu.VMEM((BPW, BLK), jnp.float32),
          pltpu.VMEM((BPW, BLK), jnp.float32),
          pltpu.VMEM((BPW, BLK), jnp.float32),
          pltpu.VMEM((BPW, BLK), jnp.float32),
          pltpu.VMEM((BPW, BLK), jnp.float32),
          pltpu.VMEM((BPW, EMB), jnp.float32),
          pltpu.SemaphoreType.DMA,
          pltpu.SemaphoreType.DMA,
          pltpu.SemaphoreType.DMA,
          pltpu.SemaphoreType.DMA,
          pltpu.SemaphoreType.DMA,
      ],
  )
  def sc_kernel(vidg_hbm, wvt_hbm, regg_hbm, cidg_hbm,
                vemb_hbm, remb_hbm, cemb_hbm,
                out_v, out_p, out_r, out_c,
                vid_v, reg_v, cid_v, wvt_v, hg_a, hg_b,
                v_rows, r_rows, c_rows, buf_a, buf_b, acc,
                sem_a, sem_b, sem_v, sem_r, sem_c):
    wid = lax.axis_index("s") * NC + lax.axis_index("c")
    base = wid * BPW
    pltpu.sync_copy(vidg_hbm.at[pl.ds(base, BPW)], vid_v)
    pltpu.sync_copy(regg_hbm.at[pl.ds(base, BPW)], reg_v)
    pltpu.sync_copy(cidg_hbm.at[pl.ds(base, BPW)], cid_v)
    pltpu.sync_copy(wvt_hbm.at[:, pl.ds(base, BPW)], wvt_v)
    cp_v = pltpu.async_copy(vemb_hbm.at[vid_v], v_rows, sem_v)
    cp_r = pltpu.async_copy(remb_hbm.at[reg_v], r_rows, sem_r)
    cp_c = pltpu.async_copy(cemb_hbm.at[cid_v], c_rows, sem_c)

    def stage_blocks(h, hg):
      for q in range(BPW // LANES):
        hg[pl.ds(q * LANES, LANES)] = jnp.right_shift(
            wvt_v[h, pl.ds(q * LANES, LANES)], 2)

    def accumulate(buf, h):
      def add_chunk(q, carry):
        qb = pl.multiple_of(q * LANES, LANES)
        off_vec = jnp.bitwise_and(wvt_v[h, pl.ds(qb, LANES)], 3) * EMB
        for r in range(LANES):
          off = pl.multiple_of(off_vec[r], EMB)
          for j in range(ROW_VREGS):
            plsc.addupdate(acc.at[qb + r, pl.ds(j * LANES, LANES)],
                           buf[qb + r, pl.ds(off + j * LANES, LANES)])
        return carry

      lax.fori_loop(0, BPW // LANES, add_chunk, 0)

    def zero_body(b, carry):
      for j in range(ROW_VREGS):
        acc[b, pl.ds(j * LANES, LANES)] = jnp.zeros((LANES,), jnp.float32)
      return carry

    lax.fori_loop(0, BPW, zero_body, 0)
    stage_blocks(0, hg_a)
    pltpu.async_copy(vemb_hbm.at[hg_a], buf_a, sem_a)
    stage_blocks(1, hg_b)
    pltpu.async_copy(vemb_hbm.at[hg_b], buf_b, sem_b)

    def pair_body(t, carry):
      h = t * 2
      pltpu.make_async_copy(vemb_hbm.at[hg_a], buf_a, sem_a).wait()
      accumulate(buf_a, h)
      stage_blocks(h + 2, hg_a)
      pltpu.async_copy(vemb_hbm.at[hg_a], buf_a, sem_a)
      pltpu.make_async_copy(vemb_hbm.at[hg_b], buf_b, sem_b).wait()
      accumulate(buf_b, h + 1)
      stage_blocks(h + 3, hg_b)
      pltpu.async_copy(vemb_hbm.at[hg_b], buf_b, sem_b)
      return carry

    lax.fori_loop(0, H // 2 - 1, pair_body, 0)
    pltpu.make_async_copy(vemb_hbm.at[hg_a], buf_a, sem_a).wait()
    accumulate(buf_a, H - 2)
    pltpu.make_async_copy(vemb_hbm.at[hg_b], buf_b, sem_b).wait()
    accumulate(buf_b, H - 1)
    cp_v.wait()
    cp_r.wait()
    cp_c.wait()
    pltpu.sync_copy(v_rows, out_v.at[pl.ds(base, BPW)])
    pltpu.sync_copy(acc, out_p.at[pl.ds(base, BPW)])
    pltpu.sync_copy(r_rows, out_r.at[pl.ds(base, BPW)])
    pltpu.sync_copy(c_rows, out_c.at[pl.ds(base, BPW)])

  return sc_kernel(vid_g, wvt, region_g, cid_g, vemb4, remb4, cemb4)


def _select_sub(blocks_ref, sub_ref):
  blocks = blocks_ref[...]
  sub = sub_ref[...]
  out = jnp.zeros((blocks.shape[0], EMB), jnp.float32)
  for s in range(4):
    piece = blocks[:, s * EMB:(s + 1) * EMB]
    out = jnp.where(sub == s, piece, out)
  return out


def _mlp_body(v_ref, vs_ref, p_ref, r_ref, rs_ref, c_ref, cs_ref,
              w0, b0, w1, b1, w2, b2, wo, bo, out_ref):
  v = _select_sub(v_ref, vs_ref)
  r = _select_sub(r_ref, rs_ref)
  c = _select_sub(c_ref, cs_ref)
  feat = jnp.concatenate([v, p_ref[...], r, c], axis=-1)
  h = jnp.maximum(
      jnp.dot(feat, w0[...], preferred_element_type=jnp.float32) + b0[...], 0.0)
  h = jnp.maximum(
      jnp.dot(h, w1[...], preferred_element_type=jnp.float32) + b1[...], 0.0)
  h = jnp.maximum(
      jnp.dot(h, w2[...], preferred_element_type=jnp.float32) + b2[...], 0.0)
  out_ref[...] = jnp.dot(h, wo[...], preferred_element_type=jnp.float32) + bo[...]


def kernel(vid, watch_vids, region, cid, vemb, remb, cemb,
           W0, b0, W1, b1, W2, b2, Wo, bo):
  vid = vid.astype(jnp.int32)
  region = region.astype(jnp.int32)
  cid = cid.astype(jnp.int32)
  wvt = watch_vids.astype(jnp.int32).T
  vemb4 = vemb.reshape(-1, BLK)
  remb4 = remb.reshape(-1, BLK)
  cemb4 = cemb.reshape(-1, BLK)
  v4, pooled, r4, c4 = _sc_gather_pool(
      vid >> 2, wvt, region >> 2, cid >> 2, vemb4, remb4, cemb4)
  logit = pl.pallas_call(
      _mlp_body,
      out_shape=jax.ShapeDtypeStruct((B, 1), jnp.float32),
  )(v4, (vid & 3).reshape(B, 1), pooled,
    r4, (region & 3).reshape(B, 1), c4, (cid & 3).reshape(B, 1),
    W0, b0.reshape(1, -1), W1, b1.reshape(1, -1),
    W2, b2.reshape(1, -1), Wo, bo.reshape(1, -1))
  return logit


# speedup vs baseline: 1.6797x; 1.0781x over previous
import functools

import jax
import jax.numpy as jnp
from jax import lax
from jax.experimental import pallas as pl
from jax.experimental.pallas import tpu as pltpu
from jax.experimental.pallas import tpu_sc as plsc

B = 4096
H = 50
EMB = 32
NC = 2
NS = 16
NW = NC * NS
BPW = B // NW
LANES = 16
ROW_VREGS = EMB // LANES


def _sc_gather_pool(vid, wvt, region, cid, vemb, remb, cemb):
  mesh = plsc.VectorSubcoreMesh(core_axis_name="c", subcore_axis_name="s")

  @functools.partial(
      pl.kernel,
      mesh=mesh,
      compiler_params=pltpu.CompilerParams(use_tc_tiling_on_sc=False),
      out_type=jax.ShapeDtypeStruct((4, B, EMB), jnp.float32),
      scratch_types=[
          pltpu.VMEM((BPW,), jnp.int32),
          pltpu.VMEM((BPW,), jnp.int32),
          pltpu.VMEM((BPW,), jnp.int32),
          pltpu.VMEM((H, BPW), jnp.int32),
          pltpu.VMEM((BPW, EMB), jnp.float32),
          pltpu.VMEM((BPW, EMB), jnp.float32),
          pltpu.VMEM((BPW, EMB), jnp.float32),
          pltpu.VMEM((BPW, EMB), jnp.float32),
          pltpu.VMEM((BPW, EMB), jnp.float32),
          pltpu.VMEM((BPW, EMB), jnp.float32),
          pltpu.SemaphoreType.DMA,
          pltpu.SemaphoreType.DMA,
          pltpu.SemaphoreType.DMA,
          pltpu.SemaphoreType.DMA,
          pltpu.SemaphoreType.DMA,
      ],
  )
  def sc_kernel(vid_hbm, wvt_hbm, region_hbm, cid_hbm,
                vemb_hbm, remb_hbm, cemb_hbm, out_hbm,
                vid_v, reg_v, cid_v, wvt_v,
                v_rows, r_rows, c_rows, buf_a, buf_b, acc,
                sem_a, sem_b, sem_v, sem_r, sem_c):
    wid = lax.axis_index("s") * NC + lax.axis_index("c")
    base = wid * BPW
    pltpu.sync_copy(vid_hbm.at[pl.ds(base, BPW)], vid_v)
    pltpu.sync_copy(region_hbm.at[pl.ds(base, BPW)], reg_v)
    pltpu.sync_copy(cid_hbm.at[pl.ds(base, BPW)], cid_v)
    pltpu.sync_copy(wvt_hbm.at[:, pl.ds(base, BPW)], wvt_v)
    cp_v = pltpu.async_copy(vemb_hbm.at[vid_v], v_rows, sem_v)
    cp_r = pltpu.async_copy(remb_hbm.at[reg_v], r_rows, sem_r)
    cp_c = pltpu.async_copy(cemb_hbm.at[cid_v], c_rows, sem_c)
    bufs = (buf_a, buf_b)
    sems = (sem_a, sem_b)
    pltpu.async_copy(vemb_hbm.at[wvt_v.at[0]], acc, sem_a).wait()
    prev = pltpu.async_copy(vemb_hbm.at[wvt_v.at[1]], bufs[1], sem_b)
    for h in range(1, H):
      nxt = None
      if h + 1 < H:
        nxt = pltpu.async_copy(vemb_hbm.at[wvt_v.at[h + 1]],
                               bufs[(h + 1) % 2], sems[(h + 1) % 2])
      prev.wait()
      buf = bufs[h % 2]

      def add_body(b, carry, buf=buf):
        for j in range(ROW_VREGS):
          plsc.addupdate(acc.at[b, pl.ds(j * LANES, LANES)],
                         buf[b, pl.ds(j * LANES, LANES)])
        return carry

      lax.fori_loop(0, BPW, add_body, 0)
      prev = nxt
    cp_v.wait()
    cp_r.wait()
    cp_c.wait()
    pltpu.sync_copy(v_rows, out_hbm.at[0, pl.ds(base, BPW)])
    pltpu.sync_copy(acc, out_hbm.at[1, pl.ds(base, BPW)])
    pltpu.sync_copy(r_rows, out_hbm.at[2, pl.ds(base, BPW)])
    pltpu.sync_copy(c_rows, out_hbm.at[3, pl.ds(base, BPW)])

  return sc_kernel(vid, wvt, region, cid, vemb, remb, cemb)


def _mlp_body(f4_ref, w0, b0, w1, b1, w2, b2, wo, bo, out_ref):
  feat = jnp.concatenate(
      [f4_ref[0], f4_ref[1], f4_ref[2], f4_ref[3]], axis=-1)
  h = jnp.maximum(
      jnp.dot(feat, w0[...], preferred_element_type=jnp.float32) + b0[...], 0.0)
  h = jnp.maximum(
      jnp.dot(h, w1[...], preferred_element_type=jnp.float32) + b1[...], 0.0)
  h = jnp.maximum(
      jnp.dot(h, w2[...], preferred_element_type=jnp.float32) + b2[...], 0.0)
  out_ref[...] = jnp.dot(h, wo[...], preferred_element_type=jnp.float32) + bo[...]


def kernel(vid, watch_vids, region, cid, vemb, remb, cemb,
           W0, b0, W1, b1, W2, b2, Wo, bo):
  vid = vid.astype(jnp.int32)
  region = region.astype(jnp.int32)
  cid = cid.astype(jnp.int32)
  wvt = watch_vids.astype(jnp.int32).T
  f4 = _sc_gather_pool(vid, wvt, region, cid, vemb, remb, cemb)
  logit = pl.pallas_call(
      _mlp_body,
      out_shape=jax.ShapeDtypeStruct((B, 1), jnp.float32),
  )(f4, W0, b0.reshape(1, -1), W1, b1.reshape(1, -1),
    W2, b2.reshape(1, -1), Wo, bo.reshape(1, -1))
  return logit
